# baseline (speedup 1.0000x reference)
import functools

import jax
import jax.numpy as jnp
from jax import lax
from jax.experimental import pallas as pl
from jax.experimental.pallas import tpu as pltpu
from jax.experimental.pallas import tpu_sc as plsc

B = 2
CD = 512
H = 224
W = 224
P = H * W
NW = 32
PP = P // NW
CC = 8
NCHUNK = CD // CC
NVEC = PP // 16


def _sc_body(x_hbm, tk_hbm, mn_hbm, buf0, buf1, m1r, m2r, m3r, m4r, sr,
             sem0, sem1):
    wid = lax.axis_index("s") * 2 + lax.axis_index("c")
    p0 = wid * PP
    row0 = 0

    @pl.loop(0, NVEC)
    def _init(j):
        sl = pl.ds(j * 16, 16)
        ninf = jnp.full((16,), -jnp.inf, jnp.float32)
        m1r[sl] = ninf
        m2r[sl] = ninf
        m3r[sl] = ninf
        m4r[sl] = ninf
        sr[sl] = jnp.zeros((16,), jnp.float32)

    bufs = (buf0, buf1)
    sems = (sem0, sem1)

    def _start(chunk, buf, sem):
        pltpu.make_async_copy(
            x_hbm.at[pl.ds(row0 + chunk * CC, CC), pl.ds(p0, PP)], buf, sem
        ).start()

    def _wait(buf, sem):
        pltpu.make_async_copy(
            x_hbm.at[pl.ds(row0, CC), pl.ds(p0, PP)], buf, sem
        ).wait()

    _start(0, buf0, sem0)
    _start(1, buf1, sem1)

    def _consume(buf):
        @pl.loop(0, NVEC // 2)
        def _pix(j):
            base = j * 32
            for off in (0, 16):
                sl = pl.ds(base + off, 16)
                m1 = m1r[sl]
                m2 = m2r[sl]
                m3 = m3r[sl]
                m4 = m4r[sl]
                s = sr[sl]
                for c in range(CC):
                    v = buf[c, sl]
                    s = s + v
                    nm1 = jnp.maximum(m1, v)
                    nm2 = jnp.maximum(m2, jnp.minimum(v, m1))
                    nm3 = jnp.maximum(m3, jnp.minimum(v, m2))
                    nm4 = jnp.maximum(m4, jnp.minimum(v, m3))
                    m1, m2, m3, m4 = nm1, nm2, nm3, nm4
                m1r[sl] = m1
                m2r[sl] = m2
                m3r[sl] = m3
                m4r[sl] = m4
                sr[sl] = s

    @pl.loop(0, NCHUNK, step=2)
    def _chunks(g):
        for b in range(2):
            gc = g + b
            _wait(bufs[b], sems[b])
            _consume(bufs[b])

            @pl.when(gc + 2 < NCHUNK)
            def _():
                _start(gc + 2, bufs[b], sems[b])

    @pl.loop(0, NVEC)
    def _fin(j):
        sl = pl.ds(j * 16, 16)
        m1r[sl] = (m1r[sl] + m2r[sl]) + (m3r[sl] + m4r[sl])
        sr[sl] = sr[sl] * jnp.float32(1.0 / CD)

    pltpu.sync_copy(m1r, tk_hbm.at[pl.ds(p0, PP)])
    pltpu.sync_copy(sr, mn_hbm.at[pl.ds(p0, PP)])


def _sc_stage(x2):
    mesh = plsc.VectorSubcoreMesh(core_axis_name="c", subcore_axis_name="s")
    return pl.kernel(
        _sc_body,
        out_type=[
            jax.ShapeDtypeStruct((P,), jnp.float32),
            jax.ShapeDtypeStruct((P,), jnp.float32),
        ],
        mesh=mesh,
        scratch_types=[
            pltpu.VMEM((CC, PP), jnp.float32),
            pltpu.VMEM((CC, PP), jnp.float32),
            pltpu.VMEM((PP,), jnp.float32),
            pltpu.VMEM((PP,), jnp.float32),
            pltpu.VMEM((PP,), jnp.float32),
            pltpu.VMEM((PP,), jnp.float32),
            pltpu.VMEM((PP,), jnp.float32),
            pltpu.SemaphoreType.DMA,
            pltpu.SemaphoreType.DMA,
        ],
        compiler_params=pltpu.CompilerParams(use_tc_tiling_on_sc=False),
    )(x2)


def _conv_body(a1_ref, a2_ref, w1_ref, b1_ref, w2_ref, b2_ref, att_ref):
    a1 = a1_ref[0]
    a2 = a2_ref[0]
    acc = jnp.full((H, W), b1_ref[0], jnp.float32)
    for ci, a in enumerate((a1, a2)):
        p = jnp.pad(a, 1)
        for di in range(3):
            for dj in range(3):
                acc = acc + w1_ref[ci, di, dj] * p[di:di + H, dj:dj + W]
    m = jnp.maximum(acc, 0.0)
    acc2 = jnp.full((H, W), b2_ref[0], jnp.float32)
    pm = jnp.pad(m, 1)
    for di in range(3):
        for dj in range(3):
            acc2 = acc2 + w2_ref[di, dj] * pm[di:di + H, dj:dj + W]
    att_ref[0] = jax.nn.sigmoid(acc2)


def _conv_stage(a1, a2, w1m, b1, w2m, b2):
    return pl.pallas_call(
        _conv_body,
        grid=(B,),
        in_specs=[
            pl.BlockSpec((1, H, W), lambda b: (b, 0, 0)),
            pl.BlockSpec((1, H, W), lambda b: (b, 0, 0)),
            pl.BlockSpec(memory_space=pltpu.SMEM),
            pl.BlockSpec(memory_space=pltpu.SMEM),
            pl.BlockSpec(memory_space=pltpu.SMEM),
            pl.BlockSpec(memory_space=pltpu.SMEM),
        ],
        out_specs=pl.BlockSpec((1, H, W), lambda b: (b, 0, 0)),
        out_shape=jax.ShapeDtypeStruct((B, H, W), jnp.float32),
    )(a1, a2, w1m, b1, w2m, b2)


def _mul_body(x_ref, att_ref, o_ref):
    o_ref[...] = x_ref[...] * att_ref[0][None, None]


def _mul_stage(x, att):
    return pl.pallas_call(
        _mul_body,
        grid=(B, 32),
        in_specs=[
            pl.BlockSpec((1, 1, 16, H, W), lambda b, c: (b, c, 0, 0, 0)),
            pl.BlockSpec((1, H, W), lambda b, c: (b, 0, 0)),
        ],
        out_specs=pl.BlockSpec((1, 1, 16, H, W), lambda b, c: (b, c, 0, 0, 0)),
        out_shape=jax.ShapeDtypeStruct((B, 32, 16, H, W), jnp.float32),
    )(x, att)


@jax.jit
def kernel(x, W1, b1, W2, b2):
    tk0, mn0 = _sc_stage(x[0].reshape(CD, P))
    tk1, mn1 = _sc_stage(x[1].reshape(CD, P))
    a1 = jnp.stack([tk0, tk1]).reshape(B, H, W)
    a2 = jnp.stack([mn0, mn1]).reshape(B, H, W)
    w1m = W1[0, :, 1]
    w2m = W2[0, 0, 1]
    att = _conv_stage(a1, a2, w1m, b1, w2m, b2)
    out5 = _mul_stage(x, att)
    return (
        out5.reshape(x.shape),
        att.reshape(B, 1, 1, H, W),
    )


# speedup vs baseline: 1.5727x; 1.5727x over previous
import functools

import jax
import jax.numpy as jnp
from jax import lax
from jax.experimental import pallas as pl
from jax.experimental.pallas import tpu as pltpu
from jax.experimental.pallas import tpu_sc as plsc

B = 2
CD = 512
H = 224
W = 224
P = H * W
NBAND = 14
RB = H // NBAND
PP = RB * W
NV = PP // 16
CC = 8
NCHUNK = CD // CC


def _sc_body(x_hbm, tk_hbm, mn_hbm, buf0, buf1, tkb, mnb,
             m1r, m2r, m3r, m4r, sr, sem0, sem1):
    wid = lax.axis_index("s") * 2 + lax.axis_index("c")

    @pl.when(wid < B * NBAND)
    def _work():
        bb = wid // NBAND
        r0 = (wid % NBAND) * RB

        @pl.loop(0, NV)
        def _init(j):
            sl = pl.ds(j * 16, 16)
            ninf = jnp.full((16,), -jnp.inf, jnp.float32)
            m1r[sl] = ninf
            m2r[sl] = ninf
            m3r[sl] = ninf
            m4r[sl] = ninf
            sr[sl] = jnp.zeros((16,), jnp.float32)

        bufs = (buf0, buf1)
        sems = (sem0, sem1)

        def _start(chunk, buf, sem):
            for c in range(CC):
                pltpu.make_async_copy(
                    x_hbm.at[bb, chunk * CC + c, pl.ds(r0, RB), :],
                    buf.at[c], sem,
                ).start()

        def _wait(buf, sem):
            for c in range(CC):
                pltpu.make_async_copy(
                    x_hbm.at[bb, c, pl.ds(r0, RB), :], buf.at[c], sem,
                ).wait()

        _start(0, buf0, sem0)
        _start(1, buf1, sem1)

        def _consume(buf):
            @pl.loop(0, RB)
            def _row(r):
                for k in range(W // 16):
                    sl = pl.ds(r * W + k * 16, 16)
                    m1 = m1r[sl]
                    m2 = m2r[sl]
                    m3 = m3r[sl]
                    m4 = m4r[sl]
                    s = sr[sl]
                    for c in range(CC):
                        v = buf[c, r, pl.ds(k * 16, 16)]
                        s = s + v
                        nm1 = jnp.maximum(m1, v)
                        nm2 = jnp.maximum(m2, jnp.minimum(v, m1))
                        nm3 = jnp.maximum(m3, jnp.minimum(v, m2))
                        nm4 = jnp.maximum(m4, jnp.minimum(v, m3))
                        m1, m2, m3, m4 = nm1, nm2, nm3, nm4
                    m1r[sl] = m1
                    m2r[sl] = m2
                    m3r[sl] = m3
                    m4r[sl] = m4
                    sr[sl] = s

        @pl.loop(0, NCHUNK, step=2)
        def _chunks(g):
            for b in range(2):
                gc = g + b
                _wait(bufs[b], sems[b])
                _consume(bufs[b])

                @pl.when(gc + 2 < NCHUNK)
                def _():
                    _start(gc + 2, bufs[b], sems[b])

        @pl.loop(0, RB)
        def _fin(r):
            for k in range(W // 16):
                sl = pl.ds(r * W + k * 16, 16)
                ksl = pl.ds(k * 16, 16)
                tkb[r, ksl] = (m1r[sl] + m2r[sl]) + (m3r[sl] + m4r[sl])
                mnb[r, ksl] = sr[sl] * jnp.float32(1.0 / CD)

        pltpu.sync_copy(tkb, tk_hbm.at[bb, pl.ds(r0, RB), :])
        pltpu.sync_copy(mnb, mn_hbm.at[bb, pl.ds(r0, RB), :])


def _sc_stage(x4):
    mesh = plsc.VectorSubcoreMesh(core_axis_name="c", subcore_axis_name="s")
    return pl.kernel(
        _sc_body,
        out_type=[
            jax.ShapeDtypeStruct((B, H, W), jnp.float32),
            jax.ShapeDtypeStruct((B, H, W), jnp.float32),
        ],
        mesh=mesh,
        scratch_types=[
            pltpu.VMEM((CC, RB, W), jnp.float32),
            pltpu.VMEM((CC, RB, W), jnp.float32),
            pltpu.VMEM((RB, W), jnp.float32),
            pltpu.VMEM((RB, W), jnp.float32),
            pltpu.VMEM((PP,), jnp.float32),
            pltpu.VMEM((PP,), jnp.float32),
            pltpu.VMEM((PP,), jnp.float32),
            pltpu.VMEM((PP,), jnp.float32),
            pltpu.VMEM((PP,), jnp.float32),
            pltpu.SemaphoreType.DMA,
            pltpu.SemaphoreType.DMA,
        ],
        compiler_params=pltpu.CompilerParams(use_tc_tiling_on_sc=True),
    )(x4)


def _conv_body(a1_ref, a2_ref, w1_ref, b1_ref, w2_ref, b2_ref, att_ref):
    a1 = a1_ref[0]
    a2 = a2_ref[0]
    acc = jnp.full((H, W), b1_ref[0], jnp.float32)
    for ci, a in enumerate((a1, a2)):
        p = jnp.pad(a, 1)
        for di in range(3):
            for dj in range(3):
                acc = acc + w1_ref[ci, di, dj] * p[di:di + H, dj:dj + W]
    m = jnp.maximum(acc, 0.0)
    acc2 = jnp.full((H, W), b2_ref[0], jnp.float32)
    pm = jnp.pad(m, 1)
    for di in range(3):
        for dj in range(3):
            acc2 = acc2 + w2_ref[di, dj] * pm[di:di + H, dj:dj + W]
    att_ref[0] = jax.nn.sigmoid(acc2)


def _conv_stage(a1, a2, w1m, b1, w2m, b2):
    return pl.pallas_call(
        _conv_body,
        grid=(B,),
        in_specs=[
            pl.BlockSpec((1, H, W), lambda b: (b, 0, 0)),
            pl.BlockSpec((1, H, W), lambda b: (b, 0, 0)),
            pl.BlockSpec(memory_space=pltpu.SMEM),
            pl.BlockSpec(memory_space=pltpu.SMEM),
            pl.BlockSpec(memory_space=pltpu.SMEM),
            pl.BlockSpec(memory_space=pltpu.SMEM),
        ],
        out_specs=pl.BlockSpec((1, H, W), lambda b: (b, 0, 0)),
        out_shape=jax.ShapeDtypeStruct((B, H, W), jnp.float32),
    )(a1, a2, w1m, b1, w2m, b2)


def _mul_body(x_ref, att_ref, o_ref):
    o_ref[...] = x_ref[...] * att_ref[0][None, None]


def _mul_stage(x, att):
    return pl.pallas_call(
        _mul_body,
        grid=(B, 32),
        in_specs=[
            pl.BlockSpec((1, 1, 16, H, W), lambda b, c: (b, c, 0, 0, 0)),
            pl.BlockSpec((1, H, W), lambda b, c: (b, 0, 0)),
        ],
        out_specs=pl.BlockSpec((1, 1, 16, H, W), lambda b, c: (b, c, 0, 0, 0)),
        out_shape=jax.ShapeDtypeStruct((B, 32, 16, H, W), jnp.float32),
    )(x, att)


@jax.jit
def kernel(x, W1, b1, W2, b2):
    x4 = x.reshape(B, CD, H, W)
    tk, mn = _sc_stage(x4)
    w1m = W1[0, :, 1]
    w2m = W2[0, 0, 1]
    att = _conv_stage(tk, mn, w1m, b1, w2m, b2)
    out5 = _mul_stage(x, att)
    return (
        out5.reshape(x.shape),
        att.reshape(B, 1, 1, H, W),
    )


# speedup vs baseline: 1.5822x; 1.0061x over previous
import functools

import jax
import jax.numpy as jnp
from jax import lax
from jax.experimental import pallas as pl
from jax.experimental.pallas import tpu as pltpu
from jax.experimental.pallas import tpu_sc as plsc

B = 2
CD = 512
H = 224
W = 224
P = H * W
NBAND = 14
RB = H // NBAND
PP = RB * W
NV = PP // 16
CC = 8
NCHUNK = CD // CC


def _sc_body(x_hbm, tk_hbm, mn_hbm, buf0, buf1, tkb, mnb,
             m1r, m2r, m3r, m4r, sr, sem0, sem1):
    wid = lax.axis_index("s") * 2 + lax.axis_index("c")

    @pl.when(wid < B * NBAND)
    def _work():
        bb = wid // NBAND
        r0 = (wid % NBAND) * RB

        @pl.loop(0, NV)
        def _init(j):
            sl = pl.ds(j * 16, 16)
            ninf = jnp.full((16,), -jnp.inf, jnp.float32)
            m1r[sl] = ninf
            m2r[sl] = ninf
            m3r[sl] = ninf
            m4r[sl] = ninf
            sr[sl] = jnp.zeros((16,), jnp.float32)

        bufs = (buf0, buf1)
        sems = (sem0, sem1)

        def _start(chunk, buf, sem):
            pltpu.make_async_copy(
                x_hbm.at[bb, pl.ds(chunk * CC, CC), pl.ds(r0, RB), :],
                buf, sem,
            ).start()

        def _wait(buf, sem):
            pltpu.make_async_copy(
                x_hbm.at[bb, pl.ds(0, CC), pl.ds(r0, RB), :], buf, sem,
            ).wait()

        _start(0, buf0, sem0)
        _start(1, buf1, sem1)

        def _consume(buf):
            @pl.loop(0, RB)
            def _row(r):
                for k in range(W // 16):
                    sl = pl.ds(r * W + k * 16, 16)
                    m1 = m1r[sl]
                    m2 = m2r[sl]
                    m3 = m3r[sl]
                    m4 = m4r[sl]
                    s = sr[sl]
                    for c in range(CC):
                        v = buf[c, r, pl.ds(k * 16, 16)]
                        s = s + v
                        nm1 = jnp.maximum(m1, v)
                        nm2 = jnp.maximum(m2, jnp.minimum(v, m1))
                        nm3 = jnp.maximum(m3, jnp.minimum(v, m2))
                        nm4 = jnp.maximum(m4, jnp.minimum(v, m3))
                        m1, m2, m3, m4 = nm1, nm2, nm3, nm4
                    m1r[sl] = m1
                    m2r[sl] = m2
                    m3r[sl] = m3
                    m4r[sl] = m4
                    sr[sl] = s

        @pl.loop(0, NCHUNK, step=2)
        def _chunks(g):
            for b in range(2):
                gc = g + b
                _wait(bufs[b], sems[b])
                _consume(bufs[b])

                @pl.when(gc + 2 < NCHUNK)
                def _():
                    _start(gc + 2, bufs[b], sems[b])

        @pl.loop(0, RB)
        def _fin(r):
            for k in range(W // 16):
                sl = pl.ds(r * W + k * 16, 16)
                ksl = pl.ds(k * 16, 16)
                tkb[r, ksl] = (m1r[sl] + m2r[sl]) + (m3r[sl] + m4r[sl])
                mnb[r, ksl] = sr[sl] * jnp.float32(1.0 / CD)

        pltpu.sync_copy(tkb, tk_hbm.at[bb, pl.ds(r0, RB), :])
        pltpu.sync_copy(mnb, mn_hbm.at[bb, pl.ds(r0, RB), :])


def _sc_stage(x4):
    mesh = plsc.VectorSubcoreMesh(core_axis_name="c", subcore_axis_name="s")
    return pl.kernel(
        _sc_body,
        out_type=[
            jax.ShapeDtypeStruct((B, H, W), jnp.float32),
            jax.ShapeDtypeStruct((B, H, W), jnp.float32),
        ],
        mesh=mesh,
        scratch_types=[
            pltpu.VMEM((CC, RB, W), jnp.float32),
            pltpu.VMEM((CC, RB, W), jnp.float32),
            pltpu.VMEM((RB, W), jnp.float32),
            pltpu.VMEM((RB, W), jnp.float32),
            pltpu.VMEM((PP,), jnp.float32),
            pltpu.VMEM((PP,), jnp.float32),
            pltpu.VMEM((PP,), jnp.float32),
            pltpu.VMEM((PP,), jnp.float32),
            pltpu.VMEM((PP,), jnp.float32),
            pltpu.SemaphoreType.DMA,
            pltpu.SemaphoreType.DMA,
        ],
        compiler_params=pltpu.CompilerParams(use_tc_tiling_on_sc=True),
    )(x4)


def _conv_body(a1_ref, a2_ref, w1_ref, b1_ref, w2_ref, b2_ref, att_ref):
    a1 = a1_ref[0]
    a2 = a2_ref[0]
    acc = jnp.full((H, W), b1_ref[0], jnp.float32)
    for ci, a in enumerate((a1, a2)):
        p = jnp.pad(a, 1)
        for di in range(3):
            for dj in range(3):
                acc = acc + w1_ref[ci, di, dj] * p[di:di + H, dj:dj + W]
    m = jnp.maximum(acc, 0.0)
    acc2 = jnp.full((H, W), b2_ref[0], jnp.float32)
    pm = jnp.pad(m, 1)
    for di in range(3):
        for dj in range(3):
            acc2 = acc2 + w2_ref[di, dj] * pm[di:di + H, dj:dj + W]
    att_ref[0] = jax.nn.sigmoid(acc2)


def _conv_stage(a1, a2, w1m, b1, w2m, b2):
    return pl.pallas_call(
        _conv_body,
        grid=(B,),
        in_specs=[
            pl.BlockSpec((1, H, W), lambda b: (b, 0, 0)),
            pl.BlockSpec((1, H, W), lambda b: (b, 0, 0)),
            pl.BlockSpec(memory_space=pltpu.SMEM),
            pl.BlockSpec(memory_space=pltpu.SMEM),
            pl.BlockSpec(memory_space=pltpu.SMEM),
            pl.BlockSpec(memory_space=pltpu.SMEM),
        ],
        out_specs=pl.BlockSpec((1, H, W), lambda b: (b, 0, 0)),
        out_shape=jax.ShapeDtypeStruct((B, H, W), jnp.float32),
    )(a1, a2, w1m, b1, w2m, b2)


def _mul_body(x_ref, att_ref, o_ref):
    o_ref[...] = x_ref[...] * att_ref[0][None, None]


def _mul_stage(x, att):
    return pl.pallas_call(
        _mul_body,
        grid=(B, 32),
        in_specs=[
            pl.BlockSpec((1, 1, 16, H, W), lambda b, c: (b, c, 0, 0, 0)),
            pl.BlockSpec((1, H, W), lambda b, c: (b, 0, 0)),
        ],
        out_specs=pl.BlockSpec((1, 1, 16, H, W), lambda b, c: (b, c, 0, 0, 0)),
        out_shape=jax.ShapeDtypeStruct((B, 32, 16, H, W), jnp.float32),
    )(x, att)


@jax.jit
def kernel(x, W1, b1, W2, b2):
    x4 = x.reshape(B, CD, H, W)
    tk, mn = _sc_stage(x4)
    w1m = W1[0, :, 1]
    w2m = W2[0, 0, 1]
    att = _conv_stage(tk, mn, w1m, b1, w2m, b2)
    out5 = _mul_stage(x, att)
    return (
        out5.reshape(x.shape),
        att.reshape(B, 1, 1, H, W),
    )


# speedup vs baseline: 2.4413x; 1.5430x over previous
import functools

import jax
import jax.numpy as jnp
from jax import lax
from jax.experimental import pallas as pl
from jax.experimental.pallas import tpu as pltpu
from jax.experimental.pallas import tpu_sc as plsc

B = 2
CD = 512
H = 224
W = 224
P = H * W
NBAND = 14
RB = H // NBAND
PP = RB * W
NV = PP // 16
CC = 8
NCHUNK = CD // CC


def _sc_body(x_hbm, tk_hbm, mn_hbm, buf0, buf1, tkb, mnb,
             m1r, m2r, m3r, m4r, sr, sem0, sem1):
    wid = lax.axis_index("s") * 2 + lax.axis_index("c")

    @pl.when(wid < B * NBAND)
    def _work():
        bb = wid // NBAND
        r0 = (wid % NBAND) * RB

        @pl.loop(0, NV)
        def _init(j):
            sl = pl.ds(j * 16, 16)
            ninf = jnp.full((16,), -jnp.inf, jnp.float32)
            m1r[sl] = ninf
            m2r[sl] = ninf
            m3r[sl] = ninf
            m4r[sl] = ninf
            sr[sl] = jnp.zeros((16,), jnp.float32)

        bufs = (buf0, buf1)
        sems = (sem0, sem1)

        def _start(chunk, buf, sem):
            pltpu.make_async_copy(
                x_hbm.at[bb, pl.ds(chunk * CC, CC), pl.ds(r0, RB), :],
                buf, sem,
            ).start()

        def _wait(buf, sem):
            pltpu.make_async_copy(
                x_hbm.at[bb, pl.ds(0, CC), pl.ds(r0, RB), :], buf, sem,
            ).wait()

        _start(0, buf0, sem0)
        _start(1, buf1, sem1)

        def _consume(buf):
            @pl.loop(0, RB)
            def _row(r):
                for k in range(W // 16):
                    sl = pl.ds(r * W + k * 16, 16)
                    m1 = m1r[sl]
                    m2 = m2r[sl]
                    m3 = m3r[sl]
                    m4 = m4r[sl]
                    s = sr[sl]
                    for c in range(CC):
                        v = buf[c, r, pl.ds(k * 16, 16)]
                        s = s + v
                        nm1 = jnp.maximum(m1, v)
                        nm2 = jnp.maximum(m2, jnp.minimum(v, m1))
                        nm3 = jnp.maximum(m3, jnp.minimum(v, m2))
                        nm4 = jnp.maximum(m4, jnp.minimum(v, m3))
                        m1, m2, m3, m4 = nm1, nm2, nm3, nm4
                    m1r[sl] = m1
                    m2r[sl] = m2
                    m3r[sl] = m3
                    m4r[sl] = m4
                    sr[sl] = s

        @pl.loop(0, NCHUNK, step=2)
        def _chunks(g):
            for b in range(2):
                gc = g + b
                _wait(bufs[b], sems[b])

                @pl.when(gc + 2 < NCHUNK)
                def _():
                    _start(gc + 2, bufs[b], sems[b])

        @pl.loop(0, RB)
        def _fin(r):
            for k in range(W // 16):
                sl = pl.ds(r * W + k * 16, 16)
                ksl = pl.ds(k * 16, 16)
                tkb[r, ksl] = (m1r[sl] + m2r[sl]) + (m3r[sl] + m4r[sl])
                mnb[r, ksl] = sr[sl] * jnp.float32(1.0 / CD)

        pltpu.sync_copy(tkb, tk_hbm.at[bb, pl.ds(r0, RB), :])
        pltpu.sync_copy(mnb, mn_hbm.at[bb, pl.ds(r0, RB), :])


def _sc_stage(x4):
    mesh = plsc.VectorSubcoreMesh(core_axis_name="c", subcore_axis_name="s")
    return pl.kernel(
        _sc_body,
        out_type=[
            jax.ShapeDtypeStruct((B, H, W), jnp.float32),
            jax.ShapeDtypeStruct((B, H, W), jnp.float32),
        ],
        mesh=mesh,
        scratch_types=[
            pltpu.VMEM((CC, RB, W), jnp.float32),
            pltpu.VMEM((CC, RB, W), jnp.float32),
            pltpu.VMEM((RB, W), jnp.float32),
            pltpu.VMEM((RB, W), jnp.float32),
            pltpu.VMEM((PP,), jnp.float32),
            pltpu.VMEM((PP,), jnp.float32),
            pltpu.VMEM((PP,), jnp.float32),
            pltpu.VMEM((PP,), jnp.float32),
            pltpu.VMEM((PP,), jnp.float32),
            pltpu.SemaphoreType.DMA,
            pltpu.SemaphoreType.DMA,
        ],
        compiler_params=pltpu.CompilerParams(use_tc_tiling_on_sc=True),
    )(x4)


def _conv_body(a1_ref, a2_ref, w1_ref, b1_ref, w2_ref, b2_ref, att_ref):
    a1 = a1_ref[0]
    a2 = a2_ref[0]
    acc = jnp.full((H, W), b1_ref[0], jnp.float32)
    for ci, a in enumerate((a1, a2)):
        p = jnp.pad(a, 1)
        for di in range(3):
            for dj in range(3):
                acc = acc + w1_ref[ci, di, dj] * p[di:di + H, dj:dj + W]
    m = jnp.maximum(acc, 0.0)
    acc2 = jnp.full((H, W), b2_ref[0], jnp.float32)
    pm = jnp.pad(m, 1)
    for di in range(3):
        for dj in range(3):
            acc2 = acc2 + w2_ref[di, dj] * pm[di:di + H, dj:dj + W]
    att_ref[0] = jax.nn.sigmoid(acc2)


def _conv_stage(a1, a2, w1m, b1, w2m, b2):
    return pl.pallas_call(
        _conv_body,
        grid=(B,),
        in_specs=[
            pl.BlockSpec((1, H, W), lambda b: (b, 0, 0)),
            pl.BlockSpec((1, H, W), lambda b: (b, 0, 0)),
            pl.BlockSpec(memory_space=pltpu.SMEM),
            pl.BlockSpec(memory_space=pltpu.SMEM),
            pl.BlockSpec(memory_space=pltpu.SMEM),
            pl.BlockSpec(memory_space=pltpu.SMEM),
        ],
        out_specs=pl.BlockSpec((1, H, W), lambda b: (b, 0, 0)),
        out_shape=jax.ShapeDtypeStruct((B, H, W), jnp.float32),
    )(a1, a2, w1m, b1, w2m, b2)


def _mul_body(x_ref, att_ref, o_ref):
    o_ref[...] = x_ref[...] * att_ref[0][None, None]


def _mul_stage(x, att):
    return pl.pallas_call(
        _mul_body,
        grid=(B, 32),
        in_specs=[
            pl.BlockSpec((1, 1, 16, H, W), lambda b, c: (b, c, 0, 0, 0)),
            pl.BlockSpec((1, H, W), lambda b, c: (b, 0, 0)),
        ],
        out_specs=pl.BlockSpec((1, 1, 16, H, W), lambda b, c: (b, c, 0, 0, 0)),
        out_shape=jax.ShapeDtypeStruct((B, 32, 16, H, W), jnp.float32),
    )(x, att)


@jax.jit
def kernel(x, W1, b1, W2, b2):
    x4 = x.reshape(B, CD, H, W)
    tk, mn = _sc_stage(x4)
    w1m = W1[0, :, 1]
    w2m = W2[0, 0, 1]
    att = _conv_stage(tk, mn, w1m, b1, w2m, b2)
    out5 = _mul_stage(x, att)
    return (
        out5.reshape(x.shape),
        att.reshape(B, 1, 1, H, W),
    )


# speedup vs baseline: 2.4527x; 1.0046x over previous
import functools

import jax
import jax.numpy as jnp
from jax import lax
from jax.experimental import pallas as pl
from jax.experimental.pallas import tpu as pltpu
from jax.experimental.pallas import tpu_sc as plsc

B = 2
CD = 512
HALF = 256
H = 224
W = 224
P = H * W
NBAND = 14
RB = H // NBAND
CC = 8
NCHUNK = HALF // CC


def _sc_body(x_hbm, m1_hbm, m2_hbm, m3_hbm, m4_hbm, s_hbm,
             buf0, buf1, m1r, m2r, m3r, m4r, sr, sem0, sem1):
    wid = lax.axis_index("s") * 2 + lax.axis_index("c")

    @pl.when(wid < B * NBAND)
    def _work():
        bb = wid // NBAND
        r0 = (wid % NBAND) * RB

        @pl.loop(0, RB)
        def _init(r):
            for k in range(W // 16):
                ksl = pl.ds(k * 16, 16)
                ninf = jnp.full((16,), -jnp.inf, jnp.float32)
                m1r[r, ksl] = ninf
                m2r[r, ksl] = ninf
                m3r[r, ksl] = ninf
                m4r[r, ksl] = ninf
                sr[r, ksl] = jnp.zeros((16,), jnp.float32)

        bufs = (buf0, buf1)
        sems = (sem0, sem1)

        def _start(chunk, buf, sem):
            pltpu.make_async_copy(
                x_hbm.at[bb, pl.ds(chunk * CC, CC), pl.ds(r0, RB), :],
                buf, sem,
            ).start()

        def _wait(buf, sem):
            pltpu.make_async_copy(
                x_hbm.at[bb, pl.ds(0, CC), pl.ds(r0, RB), :], buf, sem,
            ).wait()

        _start(0, buf0, sem0)
        _start(1, buf1, sem1)

        def _consume(buf):
            @pl.loop(0, RB)
            def _row(r):
                for k in range(W // 16):
                    ksl = pl.ds(k * 16, 16)
                    m1 = m1r[r, ksl]
                    m2 = m2r[r, ksl]
                    m3 = m3r[r, ksl]
                    m4 = m4r[r, ksl]
                    s = sr[r, ksl]
                    for c in range(CC):
                        v = buf[c, r, ksl]
                        s = s + v
                        nm1 = jnp.maximum(m1, v)
                        nm2 = jnp.maximum(m2, jnp.minimum(v, m1))
                        nm3 = jnp.maximum(m3, jnp.minimum(v, m2))
                        nm4 = jnp.maximum(m4, jnp.minimum(v, m3))
                        m1, m2, m3, m4 = nm1, nm2, nm3, nm4
                    m1r[r, ksl] = m1
                    m2r[r, ksl] = m2
                    m3r[r, ksl] = m3
                    m4r[r, ksl] = m4
                    sr[r, ksl] = s

        @pl.loop(0, NCHUNK, step=2)
        def _chunks(g):
            for b in range(2):
                gc = g + b
                _wait(bufs[b], sems[b])
                _consume(bufs[b])

                @pl.when(gc + 2 < NCHUNK)
                def _():
                    _start(gc + 2, bufs[b], sems[b])

        pltpu.sync_copy(m1r, m1_hbm.at[bb, pl.ds(r0, RB), :])
        pltpu.sync_copy(m2r, m2_hbm.at[bb, pl.ds(r0, RB), :])
        pltpu.sync_copy(m3r, m3_hbm.at[bb, pl.ds(r0, RB), :])
        pltpu.sync_copy(m4r, m4_hbm.at[bb, pl.ds(r0, RB), :])
        pltpu.sync_copy(sr, s_hbm.at[bb, pl.ds(r0, RB), :])


def _sc_stage(x4):
    mesh = plsc.VectorSubcoreMesh(core_axis_name="c", subcore_axis_name="s")
    return pl.kernel(
        _sc_body,
        out_type=[jax.ShapeDtypeStruct((B, H, W), jnp.float32)
                  for _ in range(5)],
        mesh=mesh,
        scratch_types=[
            pltpu.VMEM((CC, RB, W), jnp.float32),
            pltpu.VMEM((CC, RB, W), jnp.float32),
            pltpu.VMEM((RB, W), jnp.float32),
            pltpu.VMEM((RB, W), jnp.float32),
            pltpu.VMEM((RB, W), jnp.float32),
            pltpu.VMEM((RB, W), jnp.float32),
            pltpu.VMEM((RB, W), jnp.float32),
            pltpu.SemaphoreType.DMA,
            pltpu.SemaphoreType.DMA,
        ],
        compiler_params=pltpu.CompilerParams(use_tc_tiling_on_sc=True),
    )(x4)


def _tc_top4_body(x_ref, m1_ref, m2_ref, m3_ref, m4_ref, s_ref):
    c = pl.program_id(1)

    @pl.when(c == 0)
    def _():
        ninf = jnp.full((1, H, W), -jnp.inf, jnp.float32)
        m1_ref[...] = ninf
        m2_ref[...] = ninf
        m3_ref[...] = ninf
        m4_ref[...] = ninf
        s_ref[...] = jnp.zeros((1, H, W), jnp.float32)

    xb = x_ref[0, 0]
    m1 = m1_ref[0]
    m2 = m2_ref[0]
    m3 = m3_ref[0]
    m4 = m4_ref[0]
    s = s_ref[0]
    for i in range(16):
        v = xb[i]
        s = s + v
        nm1 = jnp.maximum(m1, v)
        nm2 = jnp.maximum(m2, jnp.minimum(v, m1))
        nm3 = jnp.maximum(m3, jnp.minimum(v, m2))
        nm4 = jnp.maximum(m4, jnp.minimum(v, m3))
        m1, m2, m3, m4 = nm1, nm2, nm3, nm4
    m1_ref[0] = m1
    m2_ref[0] = m2
    m3_ref[0] = m3
    m4_ref[0] = m4
    s_ref[0] = s


def _tc_top4_stage(x):
    nblk = (CD - HALF) // 16
    return pl.pallas_call(
        _tc_top4_body,
        grid=(B, nblk),
        in_specs=[
            pl.BlockSpec((1, 1, 16, H, W),
                         lambda b, c: (b, HALF // 16 + c, 0, 0, 0)),
        ],
        out_specs=[pl.BlockSpec((1, H, W), lambda b, c: (b, 0, 0))
                   for _ in range(5)],
        out_shape=[jax.ShapeDtypeStruct((B, H, W), jnp.float32)
                   for _ in range(5)],
    )(x)


def _conv_body(a1_ref, a2_ref, a3_ref, a4_ref, as_ref,
               c1_ref, c2_ref, c3_ref, c4_ref, cs_ref,
               w1_ref, b1_ref, w2_ref, b2_ref, att_ref):
    a1, a2, a3, a4 = a1_ref[0], a2_ref[0], a3_ref[0], a4_ref[0]
    c1, c2, c3, c4 = c1_ref[0], c2_ref[0], c3_ref[0], c4_ref[0]
    r1 = jnp.maximum(a1, c1)
    r2 = jnp.maximum(jnp.maximum(c2, a2), jnp.minimum(a1, c1))
    r3 = jnp.maximum(jnp.maximum(c3, a3),
                     jnp.maximum(jnp.minimum(a1, c2), jnp.minimum(a2, c1)))
    r4 = jnp.maximum(
        jnp.maximum(jnp.maximum(c4, a4), jnp.minimum(a2, c2)),
        jnp.maximum(jnp.minimum(a1, c3), jnp.minimum(a3, c1)))
    top4 = (r1 + r2) + (r3 + r4)
    mean = (as_ref[0] + cs_ref[0]) * jnp.float32(1.0 / CD)

    acc = jnp.full((H, W), b1_ref[0], jnp.float32)
    for ci, a in enumerate((top4, mean)):
        p = jnp.pad(a, 1)
        for di in range(3):
            for dj in range(3):
                acc = acc + w1_ref[ci, di, dj] * p[di:di + H, dj:dj + W]
    m = jnp.maximum(acc, 0.0)
    acc2 = jnp.full((H, W), b2_ref[0], jnp.float32)
    pm = jnp.pad(m, 1)
    for di in range(3):
        for dj in range(3):
            acc2 = acc2 + w2_ref[di, dj] * pm[di:di + H, dj:dj + W]
    att_ref[0] = jax.nn.sigmoid(acc2)


def _conv_stage(maps, w1m, b1, w2m, b2):
    map_spec = pl.BlockSpec((1, H, W), lambda b: (b, 0, 0))
    return pl.pallas_call(
        _conv_body,
        grid=(B,),
        in_specs=[map_spec] * 10 + [
            pl.BlockSpec(memory_space=pltpu.SMEM),
            pl.BlockSpec(memory_space=pltpu.SMEM),
            pl.BlockSpec(memory_space=pltpu.SMEM),
            pl.BlockSpec(memory_space=pltpu.SMEM),
        ],
        out_specs=pl.BlockSpec((1, H, W), lambda b: (b, 0, 0)),
        out_shape=jax.ShapeDtypeStruct((B, H, W), jnp.float32),
    )(*maps, w1m, b1, w2m, b2)


def _mul_body(x_ref, att_ref, o_ref):
    o_ref[...] = x_ref[...] * att_ref[0][None, None]


def _mul_stage(x, att):
    return pl.pallas_call(
        _mul_body,
        grid=(B, 32),
        in_specs=[
            pl.BlockSpec((1, 1, 16, H, W), lambda b, c: (b, c, 0, 0, 0)),
            pl.BlockSpec((1, H, W), lambda b, c: (b, 0, 0)),
        ],
        out_specs=pl.BlockSpec((1, 1, 16, H, W), lambda b, c: (b, c, 0, 0, 0)),
        out_shape=jax.ShapeDtypeStruct((B, 32, 16, H, W), jnp.float32),
    )(x, att)


@jax.jit
def kernel(x, W1, b1, W2, b2):
    x4 = x.reshape(B, CD, H, W)
    sc_maps = _sc_stage(x4)
    tc_maps = _tc_top4_stage(x)
    w1m = W1[0, :, 1]
    w2m = W2[0, 0, 1]
    att = _conv_stage(list(sc_maps) + list(tc_maps), w1m, b1, w2m, b2)
    out5 = _mul_stage(x, att)
    return (
        out5.reshape(x.shape),
        att.reshape(B, 1, 1, H, W),
    )
